# Initial kernel scaffold; baseline (speedup 1.0000x reference)
import functools

import jax
import jax.numpy as jnp
from jax import lax
from jax.experimental import pallas as pl
from jax.experimental.pallas import tpu as pltpu
from jax.experimental.pallas import tpu_sc as plsc

N = 10000
E = 320000
D_LAT = 128
D_EDGE = 16
D_HID = 128
D_OUT = 64

NC = 2
NS = 16
NW = NC * NS
L = 16

CHUNK = 128
NCH = E // CHUNK
ITERS = (NCH + NW - 1) // NW
ROWS_PER_TILE = N // NS
ZROWS = 125


def _pre_body(hn_ref, wa_ref, wb_ref, b1_ref, p1_ref, p2_ref):
    h = hn_ref[...]
    p1_ref[...] = jnp.dot(h, wa_ref[...], preferred_element_type=jnp.float32) + b1_ref[...]
    p2_ref[...] = jnp.dot(h, wb_ref[...], preferred_element_type=jnp.float32)


def _precompute(hn, we1a, we1b, be1):
    blk = 1000
    return pl.pallas_call(
        _pre_body,
        grid=(N // blk,),
        in_specs=[
            pl.BlockSpec((blk, D_LAT), lambda i: (i, 0)),
            pl.BlockSpec((D_LAT, D_HID), lambda i: (0, 0)),
            pl.BlockSpec((D_LAT, D_HID), lambda i: (0, 0)),
            pl.BlockSpec((1, D_HID), lambda i: (0, 0)),
        ],
        out_specs=[
            pl.BlockSpec((blk, D_HID), lambda i: (i, 0)),
            pl.BlockSpec((blk, D_HID), lambda i: (i, 0)),
        ],
        out_shape=[
            jax.ShapeDtypeStruct((N, D_HID), jnp.float32),
            jax.ShapeDtypeStruct((N, D_HID), jnp.float32),
        ],
    )(hn, we1a, we1b, be1)


def _gather_body(src_hbm, dst_hbm, p1_hbm, p2_hbm, g_hbm,
                 idx1, idx2, b1, b2, sem1, sem2):
    w = lax.axis_index("s") * NC + lax.axis_index("c")

    def step(i, carry):
        c = i * NW + w

        @pl.when(c < NCH)
        def _():
            base = c * CHUNK
            pltpu.sync_copy(src_hbm.at[pl.ds(base, CHUNK)], idx1)
            pltpu.sync_copy(dst_hbm.at[pl.ds(base, CHUNK)], idx2)
            cp1 = pltpu.async_copy(p1_hbm.at[idx1], b1, sem1)
            cp2 = pltpu.async_copy(p2_hbm.at[idx2], b2, sem2)
            cp1.wait()
            cp2.wait()

            def row(r, rcarry):
                for j in range(D_HID // L):
                    sl = pl.ds(j * L, L)
                    b1[r, sl] = b1[r, sl] + b2[r, sl]
                return rcarry

            lax.fori_loop(0, CHUNK, row, 0)
            pltpu.sync_copy(b1, g_hbm.at[pl.ds(base, CHUNK)])

        return carry

    lax.fori_loop(0, ITERS, step, 0)


def _gather(src, dst, p1, p2):
    fn = pl.kernel(
        _gather_body,
        out_type=jax.ShapeDtypeStruct((E, D_HID), jnp.float32),
        mesh=plsc.VectorSubcoreMesh(
            core_axis_name="c", subcore_axis_name="s",
            num_cores=NC, num_subcores=NS),
        scratch_types=[
            pltpu.VMEM((CHUNK,), jnp.int32),
            pltpu.VMEM((CHUNK,), jnp.int32),
            pltpu.VMEM((CHUNK, D_HID), jnp.float32),
            pltpu.VMEM((CHUNK, D_HID), jnp.float32),
            pltpu.SemaphoreType.DMA,
            pltpu.SemaphoreType.DMA,
        ],
    )
    return fn(src, dst, p1, p2)


def _edge_body(g_ref, he_ref, wc_ref, w2_ref, b2_ref, out_ref):
    x = g_ref[...] + jnp.dot(he_ref[...], wc_ref[...], preferred_element_type=jnp.float32)
    s = x * jax.nn.sigmoid(x)
    out_ref[...] = jnp.dot(s, w2_ref[...], preferred_element_type=jnp.float32) + b2_ref[...]


def _edge_mlp(g, he, we1c, we2, be2):
    blk = 1280
    return pl.pallas_call(
        _edge_body,
        grid=(E // blk,),
        in_specs=[
            pl.BlockSpec((blk, D_HID), lambda i: (i, 0)),
            pl.BlockSpec((blk, D_EDGE), lambda i: (i, 0)),
            pl.BlockSpec((D_EDGE, D_HID), lambda i: (0, 0)),
            pl.BlockSpec((D_HID, D_HID), lambda i: (0, 0)),
            pl.BlockSpec((1, D_HID), lambda i: (0, 0)),
        ],
        out_specs=pl.BlockSpec((blk, D_HID), lambda i: (i, 0)),
        out_shape=jax.ShapeDtypeStruct((E, D_HID), jnp.float32),
    )(g, he, we1c, we2, be2)


def _scatter_body(dst_hbm, hen_hbm, agg_hbm, idxv, buf, zbuf, agg_sh, sem):
    cid = lax.axis_index("c")
    sid = lax.axis_index("s")
    w = sid * NC + cid

    zeros = jnp.zeros((L,), jnp.float32)

    def zrow(r, carry):
        for j in range(D_HID // L):
            zbuf[r, pl.ds(j * L, L)] = zeros
        return carry

    lax.fori_loop(0, ZROWS, zrow, 0)
    for k in range(ROWS_PER_TILE // ZROWS):
        pltpu.sync_copy(zbuf, agg_sh.at[pl.ds(sid * ROWS_PER_TILE + k * ZROWS, ZROWS)])
    plsc.subcore_barrier()

    def step(i, carry):
        c = i * NW + w

        @pl.when(c < NCH)
        def _():
            base = c * CHUNK
            pltpu.sync_copy(dst_hbm.at[pl.ds(base, CHUNK)], idxv)
            pltpu.sync_copy(hen_hbm.at[pl.ds(base, CHUNK)], buf)
            pltpu.sync_copy(buf, agg_sh.at[idxv], add=True)

        return carry

    lax.fori_loop(0, ITERS, step, 0)
    plsc.subcore_barrier()

    for k in range(ROWS_PER_TILE // ZROWS):
        r0 = sid * ROWS_PER_TILE + k * ZROWS
        pltpu.sync_copy(agg_sh.at[pl.ds(r0, ZROWS)], agg_hbm.at[cid, pl.ds(r0, ZROWS)])


def _scatter(dst, he_new):
    fn = pl.kernel(
        _scatter_body,
        out_type=jax.ShapeDtypeStruct((NC, N, D_HID), jnp.float32),
        mesh=plsc.VectorSubcoreMesh(
            core_axis_name="c", subcore_axis_name="s",
            num_cores=NC, num_subcores=NS),
        scratch_types=[
            pltpu.VMEM((CHUNK,), jnp.int32),
            pltpu.VMEM((CHUNK, D_HID), jnp.float32),
            pltpu.VMEM((ZROWS, D_HID), jnp.float32),
            pltpu.VMEM_SHARED((N, D_HID), jnp.float32),
            pltpu.SemaphoreType.DMA,
        ],
    )
    return fn(dst, he_new)


def _node_body(hn_ref, a0_ref, a1_ref, wn1a_ref, wn1b_ref, bn1_ref,
               wn2_ref, bn2_ref, wp_ref, bp_ref, out_ref):
    agg = a0_ref[...] + a1_ref[...]
    u = (jnp.dot(hn_ref[...], wn1a_ref[...], preferred_element_type=jnp.float32)
         + jnp.dot(agg, wn1b_ref[...], preferred_element_type=jnp.float32)
         + bn1_ref[...])
    u = u * jax.nn.sigmoid(u)
    v = jnp.dot(u, wn2_ref[...], preferred_element_type=jnp.float32) + bn2_ref[...]
    out_ref[...] = jnp.dot(v, wp_ref[...], preferred_element_type=jnp.float32) + bp_ref[...]


def _node_mlp(hn, a0, a1, wn1a, wn1b, bn1, wn2, bn2, wp, bp):
    blk = 1000
    return pl.pallas_call(
        _node_body,
        grid=(N // blk,),
        in_specs=[
            pl.BlockSpec((blk, D_LAT), lambda i: (i, 0)),
            pl.BlockSpec((blk, D_HID), lambda i: (i, 0)),
            pl.BlockSpec((blk, D_HID), lambda i: (i, 0)),
            pl.BlockSpec((D_LAT, D_HID), lambda i: (0, 0)),
            pl.BlockSpec((D_HID, D_HID), lambda i: (0, 0)),
            pl.BlockSpec((1, D_HID), lambda i: (0, 0)),
            pl.BlockSpec((D_HID, D_HID), lambda i: (0, 0)),
            pl.BlockSpec((1, D_HID), lambda i: (0, 0)),
            pl.BlockSpec((D_HID, D_OUT), lambda i: (0, 0)),
            pl.BlockSpec((1, D_OUT), lambda i: (0, 0)),
        ],
        out_specs=pl.BlockSpec((blk, D_OUT), lambda i: (i, 0)),
        out_shape=jax.ShapeDtypeStruct((N, D_OUT), jnp.float32),
    )(hn, a0, a1, wn1a, wn1b, bn1, wn2, bn2, wp, bp)


def kernel(hn, he, edge_index, We1, be1, We2, be2, Wn1, bn1, Wn2, bn2, Wp, bp):
    src = edge_index[0]
    dst = edge_index[1]
    we1a = We1[:D_LAT]
    we1b = We1[D_LAT:2 * D_LAT]
    we1c = We1[2 * D_LAT:]

    p1, p2 = _precompute(hn, we1a, we1b, be1.reshape(1, -1))
    g = _gather(src, dst, p1, p2)
    he_new = _edge_mlp(g, he, we1c, We2, be2.reshape(1, -1))
    aggs = _scatter(dst, he_new)
    hn_out = _node_mlp(hn, aggs[0], aggs[1],
                       Wn1[:D_LAT], Wn1[D_LAT:], bn1.reshape(1, -1),
                       Wn2, bn2.reshape(1, -1), Wp, bp.reshape(1, -1))
    return hn_out, he_new


# speedup vs baseline: 2.8795x; 2.8795x over previous
import functools

import jax
import jax.numpy as jnp
from jax import lax
from jax.experimental import pallas as pl
from jax.experimental.pallas import tpu as pltpu
from jax.experimental.pallas import tpu_sc as plsc

N = 10000
E = 320000
D_LAT = 128
D_EDGE = 16
D_HID = 128
D_OUT = 64

NC = 2
NS = 16
NW = NC * NS
L = 16

CHUNK = 128
NCH = E // CHUNK
ITERS = (NCH + NW - 1) // NW
N_PAD = 10240
ROWS_PER_TILE = N_PAD // NS
ZROWS = CHUNK


def _pre_body(hn_ref, wa_ref, wb_ref, b1_ref, p1_ref, p2_ref):
    h = hn_ref[...]
    p1_ref[...] = jnp.dot(h, wa_ref[...], preferred_element_type=jnp.float32) + b1_ref[...]
    p2_ref[...] = jnp.dot(h, wb_ref[...], preferred_element_type=jnp.float32)


def _precompute(hn, we1a, we1b, be1):
    blk = 1000
    return pl.pallas_call(
        _pre_body,
        grid=(N // blk,),
        in_specs=[
            pl.BlockSpec((blk, D_LAT), lambda i: (i, 0)),
            pl.BlockSpec((D_LAT, D_HID), lambda i: (0, 0)),
            pl.BlockSpec((D_LAT, D_HID), lambda i: (0, 0)),
            pl.BlockSpec((1, D_HID), lambda i: (0, 0)),
        ],
        out_specs=[
            pl.BlockSpec((blk, D_HID), lambda i: (i, 0)),
            pl.BlockSpec((blk, D_HID), lambda i: (i, 0)),
        ],
        out_shape=[
            jax.ShapeDtypeStruct((N, D_HID), jnp.float32),
            jax.ShapeDtypeStruct((N, D_HID), jnp.float32),
        ],
    )(hn, we1a, we1b, be1)


def _gather_body(src_hbm, dst_hbm, p1_hbm, p2_hbm, g_hbm,
                 idx1, idx2, b1, b2, sem1, sem2):
    w = lax.axis_index("s") * NC + lax.axis_index("c")

    def step(i, carry):
        c = i * NW + w

        @pl.when(c < NCH)
        def _():
            base = c * CHUNK
            pltpu.sync_copy(src_hbm.at[pl.ds(base, CHUNK)], idx1)
            pltpu.sync_copy(dst_hbm.at[pl.ds(base, CHUNK)], idx2)
            cp1 = pltpu.async_copy(p1_hbm.at[idx1], b1, sem1)
            cp2 = pltpu.async_copy(p2_hbm.at[idx2], b2, sem2)
            cp1.wait()
            cp2.wait()

            def row(r, rcarry):
                for j in range(D_HID // L):
                    sl = pl.ds(j * L, L)
                    b1[r, sl] = b1[r, sl] + b2[r, sl]
                return rcarry

            lax.fori_loop(0, CHUNK, row, 0)
            pltpu.sync_copy(b1, g_hbm.at[pl.ds(base, CHUNK)])

        return carry

    lax.fori_loop(0, ITERS, step, 0)


def _gather(src, dst, p1, p2):
    fn = pl.kernel(
        _gather_body,
        out_type=jax.ShapeDtypeStruct((E, D_HID), jnp.float32),
        mesh=plsc.VectorSubcoreMesh(
            core_axis_name="c", subcore_axis_name="s",
            num_cores=NC, num_subcores=NS),
        scratch_types=[
            pltpu.VMEM((CHUNK,), jnp.int32),
            pltpu.VMEM((CHUNK,), jnp.int32),
            pltpu.VMEM((CHUNK, D_HID), jnp.float32),
            pltpu.VMEM((CHUNK, D_HID), jnp.float32),
            pltpu.SemaphoreType.DMA,
            pltpu.SemaphoreType.DMA,
        ],
    )
    return fn(src, dst, p1, p2)


def _edge_body(g_ref, he_ref, wc_ref, w2_ref, b2_ref, out_ref):
    x = g_ref[...] + jnp.dot(he_ref[...], wc_ref[...], preferred_element_type=jnp.float32)
    s = x * jax.nn.sigmoid(x)
    out_ref[...] = jnp.dot(s, w2_ref[...], preferred_element_type=jnp.float32) + b2_ref[...]


def _edge_mlp(g, he, we1c, we2, be2):
    blk = 1280
    return pl.pallas_call(
        _edge_body,
        grid=(E // blk,),
        in_specs=[
            pl.BlockSpec((blk, D_HID), lambda i: (i, 0)),
            pl.BlockSpec((blk, D_EDGE), lambda i: (i, 0)),
            pl.BlockSpec((D_EDGE, D_HID), lambda i: (0, 0)),
            pl.BlockSpec((D_HID, D_HID), lambda i: (0, 0)),
            pl.BlockSpec((1, D_HID), lambda i: (0, 0)),
        ],
        out_specs=pl.BlockSpec((blk, D_HID), lambda i: (i, 0)),
        out_shape=jax.ShapeDtypeStruct((E, D_HID), jnp.float32),
    )(g, he, we1c, we2, be2)


def _scatter_body(dst_hbm, hen_hbm, agg_hbm, idxv, buf, agg_sh, sem):
    cid = lax.axis_index("c")
    sid = lax.axis_index("s")
    w = sid * NC + cid

    zeros = jnp.zeros((L,), jnp.float32)

    def zrow(r, carry):
        for j in range(D_HID // L):
            buf[r, pl.ds(j * L, L)] = zeros
        return carry

    lax.fori_loop(0, ZROWS, zrow, 0)
    for k in range(ROWS_PER_TILE // ZROWS):
        pltpu.sync_copy(buf, agg_sh.at[pl.ds(sid * ROWS_PER_TILE + k * ZROWS, ZROWS)])
    plsc.subcore_barrier()

    def step(i, carry):
        c = i * NW + w

        @pl.when(c < NCH)
        def _():
            base = c * CHUNK
            pltpu.sync_copy(dst_hbm.at[pl.ds(base, CHUNK)], idxv)
            pltpu.sync_copy(hen_hbm.at[pl.ds(base, CHUNK)], buf)
            pltpu.sync_copy(buf, agg_sh.at[idxv], add=True)

        return carry

    lax.fori_loop(0, ITERS, step, 0)
    plsc.subcore_barrier()

    for k in range(ROWS_PER_TILE // ZROWS):
        r0 = sid * ROWS_PER_TILE + k * ZROWS
        pltpu.sync_copy(agg_sh.at[pl.ds(r0, ZROWS)], agg_hbm.at[cid, pl.ds(r0, ZROWS)])


def _scatter(dst, he_new):
    fn = pl.kernel(
        _scatter_body,
        out_type=jax.ShapeDtypeStruct((NC, N_PAD, D_HID), jnp.float32),
        mesh=plsc.VectorSubcoreMesh(
            core_axis_name="c", subcore_axis_name="s",
            num_cores=NC, num_subcores=NS),
        scratch_types=[
            pltpu.VMEM((CHUNK,), jnp.int32),
            pltpu.VMEM((CHUNK, D_HID), jnp.float32),
            pltpu.VMEM_SHARED((N_PAD, D_HID), jnp.float32),
            pltpu.SemaphoreType.DMA,
        ],
    )
    return fn(dst, he_new)


def _node_body(hn_ref, a0_ref, a1_ref, wn1a_ref, wn1b_ref, bn1_ref,
               wn2_ref, bn2_ref, wp_ref, bp_ref, out_ref):
    agg = a0_ref[...] + a1_ref[...]
    u = (jnp.dot(hn_ref[...], wn1a_ref[...], preferred_element_type=jnp.float32)
         + jnp.dot(agg, wn1b_ref[...], preferred_element_type=jnp.float32)
         + bn1_ref[...])
    u = u * jax.nn.sigmoid(u)
    v = jnp.dot(u, wn2_ref[...], preferred_element_type=jnp.float32) + bn2_ref[...]
    out_ref[...] = jnp.dot(v, wp_ref[...], preferred_element_type=jnp.float32) + bp_ref[...]


def _node_mlp(hn, a0, a1, wn1a, wn1b, bn1, wn2, bn2, wp, bp):
    blk = 1000
    return pl.pallas_call(
        _node_body,
        grid=(N // blk,),
        in_specs=[
            pl.BlockSpec((blk, D_LAT), lambda i: (i, 0)),
            pl.BlockSpec((blk, D_HID), lambda i: (i, 0)),
            pl.BlockSpec((blk, D_HID), lambda i: (i, 0)),
            pl.BlockSpec((D_LAT, D_HID), lambda i: (0, 0)),
            pl.BlockSpec((D_HID, D_HID), lambda i: (0, 0)),
            pl.BlockSpec((1, D_HID), lambda i: (0, 0)),
            pl.BlockSpec((D_HID, D_HID), lambda i: (0, 0)),
            pl.BlockSpec((1, D_HID), lambda i: (0, 0)),
            pl.BlockSpec((D_HID, D_OUT), lambda i: (0, 0)),
            pl.BlockSpec((1, D_OUT), lambda i: (0, 0)),
        ],
        out_specs=pl.BlockSpec((blk, D_OUT), lambda i: (i, 0)),
        out_shape=jax.ShapeDtypeStruct((N, D_OUT), jnp.float32),
    )(hn, a0, a1, wn1a, wn1b, bn1, wn2, bn2, wp, bp)


def kernel(hn, he, edge_index, We1, be1, We2, be2, Wn1, bn1, Wn2, bn2, Wp, bp):
    src = edge_index[0]
    dst = edge_index[1]
    we1a = We1[:D_LAT]
    we1b = We1[D_LAT:2 * D_LAT]
    we1c = We1[2 * D_LAT:]

    p1, p2 = _precompute(hn, we1a, we1b, be1.reshape(1, -1))
    g = _gather(src, dst, p1, p2)
    he_new = _edge_mlp(g, he, we1c, We2, be2.reshape(1, -1))
    aggs = _scatter(dst, he_new)
    hn_out = _node_mlp(hn, aggs[0], aggs[1],
                       Wn1[:D_LAT], Wn1[D_LAT:], bn1.reshape(1, -1),
                       Wn2, bn2.reshape(1, -1), Wp, bp.reshape(1, -1))
    return hn_out, he_new
